# Initial kernel scaffold; baseline (speedup 1.0000x reference)
import jax
import jax.numpy as jnp
from jax.experimental import pallas as pl


def kernel(x, edge_index, batch, w1, b1, g1, be1, m1, v1, w2, b2, w3, b3, g2, be2, m2, v2, w4, b4, lw1, lb1, lw2, lb2):
    raise NotImplementedError("write your pallas kernel here")


# speedup vs baseline: 5.0103x; 5.0103x over previous
import jax
import jax.numpy as jnp
from jax import lax
from jax.experimental import pallas as pl
from jax.experimental.pallas import tpu as pltpu
from jax.experimental.pallas import tpu_sc as plsc

_N = 50000
_E = 800000
_G = 128
_EPS = 1e-5
_R = 1000
_GRID = _N // _R
_K = 128
_NSUB = 16
_NPAD = 50048
_STRIPE = _NPAD // _NSUB


def _make_sc_segsum(F, Ttab, Ec, edge_split, shift):
    Et = Ec // _NSUB
    n_full, tail = divmod(Et, _K)
    mesh = plsc.VectorSubcoreMesh(core_axis_name="c", subcore_axis_name="s")

    def body(table, srcp, dstp, zeros, out,
             src_v, dst_v, rows, src_t, dst_t, rows_t, acc, gsem):
        c = lax.axis_index("c")
        s = lax.axis_index("s")
        cbase = c * jnp.int32(Ec if edge_split else 0)
        off = c * jnp.int32(shift)
        r0 = pl.multiple_of(s * _STRIPE, 8)
        pltpu.sync_copy(zeros.at[pl.ds(r0, _STRIPE)],
                        acc.at[pl.ds(r0, _STRIPE)])
        plsc.subcore_barrier()

        def step(j, carry):
            b = pl.multiple_of(cbase + s * Et + j * _K, 8)
            pltpu.sync_copy(srcp.at[pl.ds(b, _K)], src_v)
            if shift:
                for k in range(_K // 16):
                    src_v[pl.ds(k * 16, 16)] = src_v[pl.ds(k * 16, 16)] + off
            pltpu.async_copy(table.at[src_v], rows, gsem).wait()
            pltpu.sync_copy(dstp.at[pl.ds(b, _K)], dst_v)
            pltpu.sync_copy(rows, acc.at[dst_v], add=True)
            return carry

        lax.fori_loop(0, n_full, step, 0)
        if tail:
            b = pl.multiple_of(cbase + s * Et + n_full * _K, 8)
            pltpu.sync_copy(srcp.at[pl.ds(b, tail)], src_t)
            if shift:
                for k in range(tail // 16):
                    src_t[pl.ds(k * 16, 16)] = src_t[pl.ds(k * 16, 16)] + off
            pltpu.async_copy(table.at[src_t], rows_t, gsem).wait()
            pltpu.sync_copy(dstp.at[pl.ds(b, tail)], dst_t)
            pltpu.sync_copy(rows_t, acc.at[dst_t], add=True)
        plsc.subcore_barrier()
        pltpu.sync_copy(acc.at[pl.ds(r0, _STRIPE)],
                        out.at[c, pl.ds(r0, _STRIPE)])

    return pl.kernel(
        body,
        out_type=jax.ShapeDtypeStruct((2, _NPAD, F), jnp.float32),
        mesh=mesh,
        compiler_params=pltpu.CompilerParams(use_tc_tiling_on_sc=False),
        scratch_types=[
            pltpu.VMEM((_K,), jnp.int32),
            pltpu.VMEM((_K,), jnp.int32),
            pltpu.VMEM((_K, F), jnp.float32),
            pltpu.VMEM((max(tail, 16),), jnp.int32),
            pltpu.VMEM((max(tail, 16),), jnp.int32),
            pltpu.VMEM((max(tail, 16), F), jnp.float32),
            pltpu.VMEM_SHARED((_NPAD, F), jnp.float32),
            pltpu.SemaphoreType.DMA,
        ],
    )


def _mlp1_body(x_ref, a_ref, bf_ref, w1_ref, b1_ref, w2a_ref, b2a_ref,
               w2b_ref, b2b_ref, h1s_ref, gp1_ref):
    i = pl.program_id(0)
    z = x_ref[...] + a_ref[0] + a_ref[1]
    h = jnp.maximum(
        jnp.dot(z, w1_ref[...], preferred_element_type=jnp.float32)
        + b1_ref[...], 0.0)
    h1a = jnp.maximum(
        jnp.dot(h, w2a_ref[...], preferred_element_type=jnp.float32)
        + b2a_ref[...], 0.0)
    h1b = jnp.maximum(
        jnp.dot(h, w2b_ref[...], preferred_element_type=jnp.float32)
        + b2b_ref[...], 0.0)
    h1s_ref[0] = h1a
    h1s_ref[1] = h1b
    oh = (bf_ref[...] == lax.broadcasted_iota(jnp.int32, (_R, _G), 1))
    oh = oh.astype(jnp.float32)
    g = lax.dot_general(oh, jnp.concatenate([h1a, h1b], axis=1),
                        (((0,), (0,)), ((), ())),
                        preferred_element_type=jnp.float32)

    @pl.when(i == 0)
    def _init():
        gp1_ref[...] = jnp.zeros_like(gp1_ref)

    gp1_ref[...] += g


def _mlp2_body(h_ref, a_ref, bf_ref, gp1_ref, w3_ref, b3_ref, w4_ref, b4_ref,
               lw1_ref, lb1_ref, lw2_ref, lb2_ref, out_ref, gp2_scr):
    i = pl.program_id(0)
    z = jnp.concatenate([h_ref[0] + a_ref[0], h_ref[1] + a_ref[1]], axis=1)
    t = jnp.maximum(
        jnp.dot(z, w3_ref[...], preferred_element_type=jnp.float32)
        + b3_ref[...], 0.0)
    h2 = jnp.maximum(
        jnp.dot(t, w4_ref[...], preferred_element_type=jnp.float32)
        + b4_ref[...], 0.0)
    oh = (bf_ref[...] == lax.broadcasted_iota(jnp.int32, (_R, _G), 1))
    oh = oh.astype(jnp.float32)
    g = lax.dot_general(oh, h2, (((0,), (0,)), ((), ())),
                        preferred_element_type=jnp.float32)

    @pl.when(i == 0)
    def _init():
        gp2_scr[...] = jnp.zeros_like(gp2_scr)

    gp2_scr[...] += g

    @pl.when(i == _GRID - 1)
    def _head():
        hc = jnp.concatenate([gp1_ref[...], gp2_scr[...]], axis=1)
        u = jnp.maximum(
            jnp.dot(hc, lw1_ref[...], preferred_element_type=jnp.float32)
            + lb1_ref[...], 0.0)
        logits = jnp.dot(u, lw2_ref[...],
                         preferred_element_type=jnp.float32) + lb2_ref[...]
        m = jnp.max(logits, axis=1, keepdims=True)
        lse = m + jnp.log(jnp.sum(jnp.exp(logits - m), axis=1, keepdims=True))
        out_ref[...] = logits - lse


_sc_agg1 = _make_sc_segsum(8, _N, _E // 2, edge_split=True, shift=0)
_sc_agg2 = _make_sc_segsum(32, 2 * _N, _E, edge_split=False, shift=_N)

_const = lambda i: (0, 0)

_mlp1_call = pl.pallas_call(
    _mlp1_body,
    grid=(_GRID,),
    in_specs=[
        pl.BlockSpec((_R, 8), lambda i: (i, 0)),
        pl.BlockSpec((2, _R, 8), lambda i: (0, i, 0)),
        pl.BlockSpec((_R, 1), lambda i: (i, 0)),
        pl.BlockSpec((8, 64), _const),
        pl.BlockSpec((1, 64), _const),
        pl.BlockSpec((64, 32), _const),
        pl.BlockSpec((1, 32), _const),
        pl.BlockSpec((64, 32), _const),
        pl.BlockSpec((1, 32), _const),
    ],
    out_specs=[
        pl.BlockSpec((2, _R, 32), lambda i: (0, i, 0)),
        pl.BlockSpec((_G, 64), _const),
    ],
    out_shape=[
        jax.ShapeDtypeStruct((2, _N, 32), jnp.float32),
        jax.ShapeDtypeStruct((_G, 64), jnp.float32),
    ],
)

_mlp2_call = pl.pallas_call(
    _mlp2_body,
    grid=(_GRID,),
    in_specs=[
        pl.BlockSpec((2, _R, 32), lambda i: (0, i, 0)),
        pl.BlockSpec((2, _R, 32), lambda i: (0, i, 0)),
        pl.BlockSpec((_R, 1), lambda i: (i, 0)),
        pl.BlockSpec((_G, 64), _const),
        pl.BlockSpec((64, 64), _const),
        pl.BlockSpec((1, 64), _const),
        pl.BlockSpec((64, 64), _const),
        pl.BlockSpec((1, 64), _const),
        pl.BlockSpec((128, 128), _const),
        pl.BlockSpec((1, 128), _const),
        pl.BlockSpec((128, 128), _const),
        pl.BlockSpec((1, 128), _const),
    ],
    out_specs=pl.BlockSpec((_G, 128), _const),
    out_shape=jax.ShapeDtypeStruct((_G, 128), jnp.float32),
    scratch_shapes=[pltpu.VMEM((_G, 64), jnp.float32)],
)


def kernel(x, edge_index, batch, w1, b1, g1, be1, m1, v1, w2, b2, w3, b3,
           g2, be2, m2, v2, w4, b4, lw1, lb1, lw2, lb2):
    src = edge_index[0]
    dst = edge_index[1]
    xp = jnp.pad(x, ((0, 0), (0, 1)))

    s1 = g1 * lax.rsqrt(v1 + _EPS)
    w1f = jnp.pad(w1, ((0, 1), (0, 0))) * s1[None, :]
    b1f = ((b1 - m1) * s1 + be1).reshape(1, 64)
    s2 = g2 * lax.rsqrt(v2 + _EPS)
    w3f = w3 * s2[None, :]
    b3f = ((b3 - m2) * s2 + be2).reshape(1, 64)

    w2a, w2b = w2[:, :32], w2[:, 32:]
    b2a, b2b = b2[:32].reshape(1, 32), b2[32:].reshape(1, 32)
    b4r = b4.reshape(1, 64)
    lb1r = lb1.reshape(1, 128)
    lw2p = jnp.pad(lw2, ((0, 0), (0, 126)))
    lb2p = jnp.concatenate(
        [lb2, jnp.full((126,), -1e30, jnp.float32)]).reshape(1, 128)
    batch_i = batch.reshape(_N, 1)
    zeros8 = jnp.zeros((_NPAD, 8), jnp.float32)
    zeros32 = jnp.zeros((_NPAD, 32), jnp.float32)

    agg1 = _sc_agg1(xp, src, dst, zeros8)
    h1s, gp1 = _mlp1_call(xp, agg1, batch_i, w1f, b1f, w2a, b2a, w2b, b2b)
    agg2 = _sc_agg2(h1s.reshape(2 * _N, 32), src, dst, zeros32)
    outp = _mlp2_call(h1s, agg2, batch_i, gp1, w3f, b3f, w4, b4r,
                      lw1, lb1r, lw2p, lb2p)
    return outp[:, :2]


# speedup vs baseline: 13.0494x; 2.6045x over previous
import jax
import jax.numpy as jnp
from jax import lax
from jax.experimental import pallas as pl
from jax.experimental.pallas import tpu as pltpu
from jax.experimental.pallas import tpu_sc as plsc

_N = 50000
_E = 800000
_G = 128
_EPS = 1e-5
_R = 1000
_GRID = _N // _R
_K = 128
_NSUB = 16
_NPAD = 50048
_STRIPE = _NPAD // _NSUB


def _make_sc_segsum(F, Ttab, Ec, edge_split, shift):
    Et = Ec // _NSUB
    n_full, tail = divmod(Et, _K)
    B = next(b for b in (6, 5, 4, 3, 2) if n_full % b == 0)
    M = n_full // B
    Etot = Ec * 2 if edge_split else Ec
    mesh = plsc.VectorSubcoreMesh(core_axis_name="c", subcore_axis_name="s")

    def body(table, srcp, dstp, zeros, out, *rest):
        srcs = rest[0:B]
        dsts = rest[B:2 * B]
        rows = rest[2 * B:3 * B]
        src_t, dst_t, rows_t, acc, isem, gsem, ssem, tsem = rest[3 * B:]
        c = lax.axis_index("c")
        s = lax.axis_index("s")
        cbase = c * jnp.int32(Ec if edge_split else 0)
        off = c * jnp.int32(shift)
        r0 = pl.multiple_of(s * _STRIPE, 8)

        def eoff(g):
            return pl.multiple_of(
                jnp.minimum(cbase + s * Et + g * _K, Etot - _K), 8)

        def issue_idx(b, g):
            o = eoff(g)
            pltpu.async_copy(srcp.at[pl.ds(o, _K)], srcs[b], isem.at[b])
            pltpu.async_copy(dstp.at[pl.ds(o, _K)], dsts[b], isem.at[b])

        def wait_idx(b):
            pltpu.make_async_copy(srcp.at[pl.ds(0, _K)], srcs[b],
                                  isem.at[b]).wait()
            pltpu.make_async_copy(dstp.at[pl.ds(0, _K)], dsts[b],
                                  isem.at[b]).wait()

        def bump(b):
            if shift:
                for k in range(_K // 16):
                    srcs[b][pl.ds(k * 16, 16)] = (
                        srcs[b][pl.ds(k * 16, 16)] + off)

        pltpu.sync_copy(zeros.at[pl.ds(r0, _STRIPE)],
                        acc.at[pl.ds(r0, _STRIPE)])
        for b in range(B):
            issue_idx(b, b)
        plsc.subcore_barrier()

        for b in range(B):
            wait_idx(b)
            bump(b)
            pltpu.async_copy(table.at[srcs[b]], rows[b], gsem.at[b])
        for b in range(B):
            pltpu.make_async_copy(table.at[srcs[b]], rows[b],
                                  gsem.at[b]).wait()
            pltpu.async_copy(rows[b], acc.at[dsts[b]], ssem.at[b], add=True)
            issue_idx(b, B + b)

        def step(m, carry):
            for b in range(B):
                g = m * B + b
                wait_idx(b)
                bump(b)
                pltpu.make_async_copy(rows[b], acc.at[dsts[b]],
                                      ssem.at[b]).wait()
                pltpu.async_copy(table.at[srcs[b]], rows[b], gsem.at[b])
            for b in range(B):
                g = m * B + b
                pltpu.make_async_copy(table.at[srcs[b]], rows[b],
                                      gsem.at[b]).wait()
                pltpu.async_copy(rows[b], acc.at[dsts[b]], ssem.at[b],
                                 add=True)
                issue_idx(b, g + B)
            return carry

        lax.fori_loop(1, M, step, 0)
        for b in range(B):
            pltpu.make_async_copy(rows[b], acc.at[dsts[b]], ssem.at[b]).wait()
            wait_idx(b)
        if tail:
            bt = pl.multiple_of(cbase + s * Et + n_full * _K, 8)
            pltpu.sync_copy(srcp.at[pl.ds(bt, tail)], src_t)
            if shift:
                for k in range(tail // 16):
                    src_t[pl.ds(k * 16, 16)] = src_t[pl.ds(k * 16, 16)] + off
            pltpu.async_copy(table.at[src_t], rows_t, tsem).wait()
            pltpu.sync_copy(dstp.at[pl.ds(bt, tail)], dst_t)
            pltpu.sync_copy(rows_t, acc.at[dst_t], add=True)
        plsc.subcore_barrier()
        pltpu.sync_copy(acc.at[pl.ds(r0, _STRIPE)],
                        out.at[c, pl.ds(r0, _STRIPE)])

    return pl.kernel(
        body,
        out_type=jax.ShapeDtypeStruct((2, _NPAD, F), jnp.float32),
        mesh=mesh,
        compiler_params=pltpu.CompilerParams(use_tc_tiling_on_sc=False),
        scratch_types=(
            [pltpu.VMEM((_K,), jnp.int32) for _ in range(2 * B)]
            + [pltpu.VMEM((_K, F), jnp.float32) for _ in range(B)]
            + [
                pltpu.VMEM((max(tail, 16),), jnp.int32),
                pltpu.VMEM((max(tail, 16),), jnp.int32),
                pltpu.VMEM((max(tail, 16), F), jnp.float32),
                pltpu.VMEM_SHARED((_NPAD, F), jnp.float32),
                pltpu.SemaphoreType.DMA((B,)),
                pltpu.SemaphoreType.DMA((B,)),
                pltpu.SemaphoreType.DMA((B,)),
                pltpu.SemaphoreType.DMA,
            ]
        ),
    )


def _mlp1_body(x_ref, a_ref, bf_ref, w1_ref, b1_ref, w2a_ref, b2a_ref,
               w2b_ref, b2b_ref, h1s_ref, gp1_ref):
    i = pl.program_id(0)
    z = x_ref[...] + a_ref[0] + a_ref[1]
    h = jnp.maximum(
        jnp.dot(z, w1_ref[...], preferred_element_type=jnp.float32)
        + b1_ref[...], 0.0)
    h1a = jnp.maximum(
        jnp.dot(h, w2a_ref[...], preferred_element_type=jnp.float32)
        + b2a_ref[...], 0.0)
    h1b = jnp.maximum(
        jnp.dot(h, w2b_ref[...], preferred_element_type=jnp.float32)
        + b2b_ref[...], 0.0)
    h1s_ref[0] = h1a
    h1s_ref[1] = h1b
    oh = (bf_ref[...] == lax.broadcasted_iota(jnp.int32, (_R, _G), 1))
    oh = oh.astype(jnp.float32)
    g = lax.dot_general(oh, jnp.concatenate([h1a, h1b], axis=1),
                        (((0,), (0,)), ((), ())),
                        preferred_element_type=jnp.float32)

    @pl.when(i == 0)
    def _init():
        gp1_ref[...] = jnp.zeros_like(gp1_ref)

    gp1_ref[...] += g


def _mlp2_body(h_ref, a_ref, bf_ref, gp1_ref, w3_ref, b3_ref, w4_ref, b4_ref,
               lw1_ref, lb1_ref, lw2_ref, lb2_ref, out_ref, gp2_scr):
    i = pl.program_id(0)
    z = jnp.concatenate([h_ref[0] + a_ref[0], h_ref[1] + a_ref[1]], axis=1)
    t = jnp.maximum(
        jnp.dot(z, w3_ref[...], preferred_element_type=jnp.float32)
        + b3_ref[...], 0.0)
    h2 = jnp.maximum(
        jnp.dot(t, w4_ref[...], preferred_element_type=jnp.float32)
        + b4_ref[...], 0.0)
    oh = (bf_ref[...] == lax.broadcasted_iota(jnp.int32, (_R, _G), 1))
    oh = oh.astype(jnp.float32)
    g = lax.dot_general(oh, h2, (((0,), (0,)), ((), ())),
                        preferred_element_type=jnp.float32)

    @pl.when(i == 0)
    def _init():
        gp2_scr[...] = jnp.zeros_like(gp2_scr)

    gp2_scr[...] += g

    @pl.when(i == _GRID - 1)
    def _head():
        hc = jnp.concatenate([gp1_ref[...], gp2_scr[...]], axis=1)
        u = jnp.maximum(
            jnp.dot(hc, lw1_ref[...], preferred_element_type=jnp.float32)
            + lb1_ref[...], 0.0)
        logits = jnp.dot(u, lw2_ref[...],
                         preferred_element_type=jnp.float32) + lb2_ref[...]
        m = jnp.max(logits, axis=1, keepdims=True)
        lse = m + jnp.log(jnp.sum(jnp.exp(logits - m), axis=1, keepdims=True))
        out_ref[...] = logits - lse


_sc_agg1 = _make_sc_segsum(8, _N, _E // 2, edge_split=True, shift=0)
_sc_agg2 = _make_sc_segsum(32, 2 * _N, _E, edge_split=False, shift=_N)

_const = lambda i: (0, 0)

_mlp1_call = pl.pallas_call(
    _mlp1_body,
    grid=(_GRID,),
    in_specs=[
        pl.BlockSpec((_R, 8), lambda i: (i, 0)),
        pl.BlockSpec((2, _R, 8), lambda i: (0, i, 0)),
        pl.BlockSpec((_R, 1), lambda i: (i, 0)),
        pl.BlockSpec((8, 64), _const),
        pl.BlockSpec((1, 64), _const),
        pl.BlockSpec((64, 32), _const),
        pl.BlockSpec((1, 32), _const),
        pl.BlockSpec((64, 32), _const),
        pl.BlockSpec((1, 32), _const),
    ],
    out_specs=[
        pl.BlockSpec((2, _R, 32), lambda i: (0, i, 0)),
        pl.BlockSpec((_G, 64), _const),
    ],
    out_shape=[
        jax.ShapeDtypeStruct((2, _N, 32), jnp.float32),
        jax.ShapeDtypeStruct((_G, 64), jnp.float32),
    ],
)

_mlp2_call = pl.pallas_call(
    _mlp2_body,
    grid=(_GRID,),
    in_specs=[
        pl.BlockSpec((2, _R, 32), lambda i: (0, i, 0)),
        pl.BlockSpec((2, _R, 32), lambda i: (0, i, 0)),
        pl.BlockSpec((_R, 1), lambda i: (i, 0)),
        pl.BlockSpec((_G, 64), _const),
        pl.BlockSpec((64, 64), _const),
        pl.BlockSpec((1, 64), _const),
        pl.BlockSpec((64, 64), _const),
        pl.BlockSpec((1, 64), _const),
        pl.BlockSpec((128, 128), _const),
        pl.BlockSpec((1, 128), _const),
        pl.BlockSpec((128, 128), _const),
        pl.BlockSpec((1, 128), _const),
    ],
    out_specs=pl.BlockSpec((_G, 128), _const),
    out_shape=jax.ShapeDtypeStruct((_G, 128), jnp.float32),
    scratch_shapes=[pltpu.VMEM((_G, 64), jnp.float32)],
)


def kernel(x, edge_index, batch, w1, b1, g1, be1, m1, v1, w2, b2, w3, b3,
           g2, be2, m2, v2, w4, b4, lw1, lb1, lw2, lb2):
    src = edge_index[0]
    dst = edge_index[1]
    xp = jnp.pad(x, ((0, 0), (0, 1)))

    s1 = g1 * lax.rsqrt(v1 + _EPS)
    w1f = jnp.pad(w1, ((0, 1), (0, 0))) * s1[None, :]
    b1f = ((b1 - m1) * s1 + be1).reshape(1, 64)
    s2 = g2 * lax.rsqrt(v2 + _EPS)
    w3f = w3 * s2[None, :]
    b3f = ((b3 - m2) * s2 + be2).reshape(1, 64)

    w2a, w2b = w2[:, :32], w2[:, 32:]
    b2a, b2b = b2[:32].reshape(1, 32), b2[32:].reshape(1, 32)
    b4r = b4.reshape(1, 64)
    lb1r = lb1.reshape(1, 128)
    lw2p = jnp.pad(lw2, ((0, 0), (0, 126)))
    lb2p = jnp.concatenate(
        [lb2, jnp.full((126,), -1e30, jnp.float32)]).reshape(1, 128)
    batch_i = batch.reshape(_N, 1)
    zeros8 = jnp.zeros((_NPAD, 8), jnp.float32)
    zeros32 = jnp.zeros((_NPAD, 32), jnp.float32)

    agg1 = _sc_agg1(xp, src, dst, zeros8)
    h1s, gp1 = _mlp1_call(xp, agg1, batch_i, w1f, b1f, w2a, b2a, w2b, b2b)
    agg2 = _sc_agg2(h1s.reshape(2 * _N, 32), src, dst, zeros32)
    outp = _mlp2_call(h1s, agg2, batch_i, gp1, w3f, b3f, w4, b4r,
                      lw1, lb1r, lw2p, lb2p)
    return outp[:, :2]
